# baseline (speedup 1.0000x reference)
import functools

import jax
import jax.numpy as jnp
from jax import lax
from jax.experimental import pallas as pl
from jax.experimental.pallas import tpu as pltpu
from jax.experimental.pallas import tpu_sc as plsc

_NC = 2
_NS = 16
_NW = _NC * _NS
_LANES = 16
_UNROLL = 8

_B = 4
_S = 4096
_E = 1024
_R = 8
_ROWS_PER_W = _S // _NW
_T = _ROWS_PER_W // _R


def _sc_body(tok_hbm, pos_hbm, out_hbm,
             pos0, pos1, tok0, tok1, tok2,
             isem0, isem1, osem0, osem1):
    wid = lax.axis_index("s") * _NC + lax.axis_index("c")
    s0 = wid * _ROWS_PER_W
    pos_bufs = (pos0, pos1)
    tok_sets = (tok0, tok1, tok2)
    isems = (isem0, isem1)
    osems = (osem0, osem1)

    def issue_in(t):
        row = s0 + t * _R
        sem = isems[t % 2]
        tset = tok_sets[t % 3]
        return [
            pltpu.async_copy(pos_hbm.at[pl.ds(row, _R), :],
                             pos_bufs[t % 2], sem),
            pltpu.async_copy(tok_hbm.at[:, pl.ds(row, _R), :], tset, sem),
        ]

    def issue_out(t):
        row = s0 + t * _R
        tset = tok_sets[t % 3]
        return [pltpu.async_copy(
            tset, out_hbm.at[:, pl.ds(row, _R), :], osems[t % 2])]

    def compute(t):
        pos_v = pos_bufs[t % 2]
        tset = tok_sets[t % 3]

        @plsc.parallel_loop(0, _R)
        def _row_loop(r):
            @plsc.parallel_loop(0, _E, step=_LANES, unroll=_UNROLL)
            def _add_loop(c):
                sl = pl.ds(c, _LANES)
                p = pos_v[r, sl]
                for b in range(_B):
                    plsc.addupdate(tset.at[b, r, sl], p)

    in_h = {0: issue_in(0)}
    out_h = {}
    for t in range(_T):
        if t >= 2:
            for h in out_h[t - 2]:
                h.wait()
        if t + 1 < _T:
            in_h[t + 1] = issue_in(t + 1)
        for h in in_h[t]:
            h.wait()
        compute(t)
        out_h[t] = issue_out(t)
    for h in out_h[_T - 2]:
        h.wait()
    for h in out_h[_T - 1]:
        h.wait()


def kernel(token_embedding, pos_table):
    B, S, E = token_embedding.shape

    mesh = plsc.VectorSubcoreMesh(core_axis_name="c", subcore_axis_name="s")
    run = functools.partial(
        pl.kernel,
        out_type=jax.ShapeDtypeStruct((B, S, E), jnp.float32),
        mesh=mesh,
        scratch_types=(
            [pltpu.VMEM((_R, _E), jnp.float32)] * 2
            + [pltpu.VMEM((_B, _R, _E), jnp.float32)] * 3
            + [pltpu.SemaphoreType.DMA] * 4
        ),
    )(_sc_body)
    return run(token_embedding, pos_table)


# speedup vs baseline: 1.0200x; 1.0200x over previous
import functools

import jax
import jax.numpy as jnp
from jax import lax
from jax.experimental import pallas as pl
from jax.experimental.pallas import tpu as pltpu
from jax.experimental.pallas import tpu_sc as plsc

_NC = 2
_NS = 16
_NW = _NC * _NS
_LANES = 16
_UNROLL = 2

_B = 4
_S = 4096
_E = 1024
_R = 8
_ROWS_PER_W = _S // _NW
_T = _ROWS_PER_W // _R


def _sc_body(tok_hbm, pos_hbm, out_hbm,
             pos0, pos1, tok0, tok1, tok2,
             isem0, isem1, osem0, osem1):
    wid = lax.axis_index("s") * _NC + lax.axis_index("c")
    s0 = wid * _ROWS_PER_W
    pos_bufs = (pos0, pos1)
    tok_sets = (tok0, tok1, tok2)
    isems = (isem0, isem1)
    osems = (osem0, osem1)

    def issue_in(t):
        row = s0 + t * _R
        sem = isems[t % 2]
        tset = tok_sets[t % 3]
        return [
            pltpu.async_copy(pos_hbm.at[pl.ds(row, _R), :],
                             pos_bufs[t % 2], sem),
            pltpu.async_copy(tok_hbm.at[:, pl.ds(row, _R), :], tset, sem),
        ]

    def issue_out(t):
        row = s0 + t * _R
        tset = tok_sets[t % 3]
        return [pltpu.async_copy(
            tset, out_hbm.at[:, pl.ds(row, _R), :], osems[t % 2])]

    def compute(t):
        pos_v = pos_bufs[t % 2]
        tset = tok_sets[t % 3]

        @plsc.parallel_loop(0, _R)
        def _row_loop(r):
            @plsc.parallel_loop(0, _E, step=_LANES, unroll=_UNROLL)
            def _add_loop(c):
                sl = pl.ds(c, _LANES)
                p = pos_v[r, sl]
                for b in range(_B):
                    plsc.addupdate(tset.at[b, r, sl], p)

    in_h = {0: issue_in(0)}
    out_h = {}
    for t in range(_T):
        if t >= 2:
            for h in out_h[t - 2]:
                h.wait()
        if t + 1 < _T:
            in_h[t + 1] = issue_in(t + 1)
        for h in in_h[t]:
            h.wait()
        compute(t)
        out_h[t] = issue_out(t)
    for h in out_h[_T - 2]:
        h.wait()
    for h in out_h[_T - 1]:
        h.wait()


def kernel(token_embedding, pos_table):
    B, S, E = token_embedding.shape

    mesh = plsc.VectorSubcoreMesh(core_axis_name="c", subcore_axis_name="s")
    run = functools.partial(
        pl.kernel,
        out_type=jax.ShapeDtypeStruct((B, S, E), jnp.float32),
        mesh=mesh,
        scratch_types=(
            [pltpu.VMEM((_R, _E), jnp.float32)] * 2
            + [pltpu.VMEM((_B, _R, _E), jnp.float32)] * 3
            + [pltpu.SemaphoreType.DMA] * 4
        ),
    )(_sc_body)
    return run(token_embedding, pos_table)
